# baseline (speedup 1.0000x reference)
import jax
import jax.numpy as jnp
from jax.experimental import pallas as pl
from jax.experimental.pallas import tpu as pltpu

THRESH = 50.0
KWTA = 16

T = 16
K = 1024
H = 256
L = 256
HBLK = 8
KC = H // HBLK


def _column_kernel(a_ref, w1_ref, w2_ref, out_ref, pot_ref):
    c = pl.program_id(0)

    @pl.when(c == 0)
    def _init():
        pot_ref[...] = jnp.zeros_like(pot_ref)

    for half, w_ref in ((0, w1_ref), (1, w2_ref)):
        acc = jnp.zeros((T, K // 2), jnp.float32)
        for hh in range(HBLK):
            acc += jax.lax.dot_general(
                a_ref[:, hh, :], w_ref[:, hh, :],
                dimension_numbers=(((1,), (1,)), ((), ())),
                preferred_element_type=jnp.float32,
            )
        pot_ref[:, half * (K // 2):(half + 1) * (K // 2)] += acc

    @pl.when(c == KC - 1)
    def _epilogue():
        raw = pot_ref[...]
        pot = jnp.where(raw > THRESH, raw, 0.0)
        spikes = (pot > 0.0).astype(jnp.float32)
        vmax = jnp.max(pot) * T
        totals = jnp.sum(pot + spikes * vmax, axis=0, keepdims=True)

        iota = jax.lax.broadcasted_iota(jnp.int32, (1, K), 1)
        mask = jnp.zeros((1, K), jnp.float32)
        work = totals
        for _ in range(KWTA):
            m = jnp.max(work)
            idx = jnp.min(jnp.where(work == m, iota, K))
            won = (m > 0.0).astype(jnp.float32)
            sel = (iota == idx)
            mask = mask + jnp.where(sel, won, 0.0)
            work = jnp.where(sel, -jnp.inf, work)

        out_ref[...] = spikes * mask


@jax.jit
def kernel(rec_field, W):
    A = jnp.squeeze(rec_field, 1)
    Wm = jnp.squeeze(W, 1)
    W1, W2 = Wm[:K // 2], Wm[K // 2:]

    spikes_masked = pl.pallas_call(
        _column_kernel,
        grid=(KC,),
        in_specs=[
            pl.BlockSpec((T, HBLK, L), lambda c: (0, c, 0)),
            pl.BlockSpec((K // 2, HBLK, L), lambda c: (0, c, 0)),
            pl.BlockSpec((K // 2, HBLK, L), lambda c: (0, c, 0)),
        ],
        out_specs=pl.BlockSpec((T, K), lambda c: (0, 0)),
        out_shape=jax.ShapeDtypeStruct((T, K), jnp.float32),
        scratch_shapes=[pltpu.VMEM((T, K), jnp.float32)],
    )(A, W1, W2)

    return spikes_masked.reshape(T, K, 1, 1)


# speedup vs baseline: 1.8273x; 1.8273x over previous
import jax
import jax.numpy as jnp
from jax.experimental import pallas as pl
from jax.experimental.pallas import tpu as pltpu

THRESH = 50.0
KWTA = 16

T = 16
K = 1024
H = 256
L = 256
KO = 2
OBLK = K // KO
HBLK = 16
KC = H // HBLK


def _column_kernel(a_ref, w_ref, out_ref, pot_ref):
    o = pl.program_id(0)
    c = pl.program_id(1)

    @pl.when(jnp.logical_and(o == 0, c == 0))
    def _init():
        pot_ref[...] = jnp.zeros_like(pot_ref)

    acc = jnp.zeros((T, OBLK), jnp.float32)
    for hh in range(HBLK):
        acc += jax.lax.dot_general(
            a_ref[:, hh, :], w_ref[:, hh, :],
            dimension_numbers=(((1,), (1,)), ((), ())),
            preferred_element_type=jnp.float32,
        )
    pot_ref[:, pl.ds(o * OBLK, OBLK)] += acc

    @pl.when(jnp.logical_and(o == KO - 1, c == KC - 1))
    def _epilogue():
        raw = pot_ref[...]
        pot = jnp.where(raw > THRESH, raw, 0.0)
        spikes = (pot > 0.0).astype(jnp.float32)
        vmax = jnp.max(pot) * T
        totals = jnp.sum(pot + spikes * vmax, axis=0, keepdims=True)

        iota = jax.lax.broadcasted_iota(jnp.int32, (1, K), 1)
        mask = jnp.zeros((1, K), jnp.float32)
        work = totals
        for _ in range(KWTA):
            m = jnp.max(work)
            idx = jnp.min(jnp.where(work == m, iota, K))
            won = (m > 0.0).astype(jnp.float32)
            sel = (iota == idx)
            mask = mask + jnp.where(sel, won, 0.0)
            work = jnp.where(sel, -jnp.inf, work)

        out_ref[...] = spikes * mask


@jax.jit
def kernel(rec_field, W):
    A = jnp.squeeze(rec_field, 1)
    Wm = jnp.squeeze(W, 1)

    spikes_masked = pl.pallas_call(
        _column_kernel,
        grid=(KO, KC),
        in_specs=[
            pl.BlockSpec((T, HBLK, L), lambda o, c: (0, c, 0)),
            pl.BlockSpec((OBLK, HBLK, L), lambda o, c: (o, c, 0)),
        ],
        out_specs=pl.BlockSpec((T, K), lambda o, c: (0, 0)),
        out_shape=jax.ShapeDtypeStruct((T, K), jnp.float32),
        scratch_shapes=[pltpu.VMEM((T, K), jnp.float32)],
    )(A, Wm)

    return spikes_masked.reshape(T, K, 1, 1)


# speedup vs baseline: 2.1053x; 1.1522x over previous
import functools

import jax
import jax.numpy as jnp
from jax import lax
from jax.experimental import pallas as pl
from jax.experimental.pallas import tpu as pltpu
from jax.experimental.pallas import tpu_sc as plsc

THRESH = 50.0
KWTA = 16

T = 16
K = 1024
H = 256
L = 256
HBLK = 8
KC = H // HBLK

SC_SUB = 8
CH_PER = K // SC_SUB
NG = CH_PER // 16
NCAND = SC_SUB * KWTA
NCV = NCAND // 16
BIG = jnp.int32(1 << 30)


def _matmul_kernel(a_ref, w_ref, pot_ref, acc_ref):
    c = pl.program_id(0)

    @pl.when(c == 0)
    def _init():
        acc_ref[...] = jnp.zeros_like(acc_ref)

    acc = jnp.zeros((T, K), jnp.float32)
    for hh in range(HBLK):
        acc += jax.lax.dot_general(
            a_ref[:, hh, :], w_ref[:, hh, :],
            dimension_numbers=(((1,), (1,)), ((), ())),
            preferred_element_type=jnp.float32,
        )
    acc_ref[...] += acc

    @pl.when(c == KC - 1)
    def _threshold():
        v = acc_ref[...]
        pot_ref[...] = jnp.where(v > THRESH, v, 0.0)


def _sc_epilogue_body(pot_hbm, chidx_hbm, out_hbm, pot_v, chidx_v, totals_v,
                      mx_v, lv_v, li_v, out_v, cval_v, cidx_v, scr_f, scr_i,
                      mx_sh, cval_sh, cidx_sh):
    s = lax.axis_index("s")
    act = jnp.logical_and(lax.axis_index("c") == 0, s < SC_SUB)
    base = s * CH_PER
    lane = lax.broadcasted_iota(jnp.int32, (16,), 0)
    zidx = lane * 0
    ninf = jnp.float32(-jnp.inf)
    bigv = jnp.full((16,), BIG, jnp.int32)

    def splat_max(vregs):
        m = vregs[0]
        for vv in vregs[1:]:
            m = jnp.maximum(m, vv)
        for k in (1, 2, 4, 8):
            scr_f[...] = m
            m = jnp.maximum(m, plsc.load_gather(scr_f, [lane ^ k]))
        return m

    def splat_min_i32(b):
        for k in (1, 2, 4, 8):
            scr_i[...] = b
            b = jnp.minimum(b, plsc.load_gather(scr_i, [lane ^ k]))
        return b

    @pl.when(act)
    def _phase1():
        pltpu.sync_copy(pot_hbm.at[:, pl.ds(base, CH_PER)], pot_v)
        pltpu.sync_copy(chidx_hbm.at[pl.ds(base, CH_PER)], chidx_v)

        mx = jnp.zeros((16,), jnp.float32)
        for g in range(NG):
            for t in range(T):
                mx = jnp.maximum(mx, pot_v[t, pl.ds(g * 16, 16)])
        mx_v[...] = mx
        pltpu.sync_copy(mx_v, mx_sh.at[pl.ds(s * 16, 16)])

    plsc.subcore_barrier()

    @pl.when(act)
    def _phase2():
        pltpu.sync_copy(mx_sh, cval_v.at[pl.ds(0, SC_SUB * 16)])
        vsv = splat_max([cval_v[pl.ds(i * 16, 16)]
                         for i in range(SC_SUB)]) * jnp.float32(T)

        for g in range(NG):
            tot = jnp.zeros((16,), jnp.float32)
            for t in range(T):
                x = pot_v[t, pl.ds(g * 16, 16)]
                tot = tot + jnp.where(x > 0.0, x + vsv, x)
            totals_v[pl.ds(g * 16, 16)] = tot

        lv = jnp.full((16,), ninf, jnp.float32)
        li = jnp.full((16,), BIG, jnp.int32)
        for j in range(KWTA):
            msv = splat_max([totals_v[pl.ds(g * 16, 16)] for g in range(NG)])
            bi = bigv
            for g in range(NG):
                w = totals_v[pl.ds(g * 16, 16)]
                gx = chidx_v[pl.ds(g * 16, 16)]
                bi = jnp.minimum(bi, jnp.where(w == msv, gx, bigv))
            miv = splat_min_i32(bi)
            lv = jnp.where(lane == j, msv, lv)
            li = jnp.where(lane == j, miv, li)
            for g in range(NG):
                w = totals_v[pl.ds(g * 16, 16)]
                gx = chidx_v[pl.ds(g * 16, 16)]
                totals_v[pl.ds(g * 16, 16)] = jnp.where(gx == miv, ninf, w)
        lv_v[...] = lv
        li_v[...] = li
        pltpu.sync_copy(lv_v, cval_sh.at[pl.ds(s * 16, 16)])
        pltpu.sync_copy(li_v, cidx_sh.at[pl.ds(s * 16, 16)])

    plsc.subcore_barrier()

    @pl.when(act)
    def _phase3():
        pltpu.sync_copy(cval_sh, cval_v)
        pltpu.sync_copy(cidx_sh, cidx_v)
        masks = [jnp.zeros((16,), jnp.float32) for _ in range(NG)]
        for j in range(KWTA):
            msv = splat_max([cval_v[pl.ds(i * 16, 16)] for i in range(NCV)])
            bi = bigv
            for i in range(NCV):
                w = cval_v[pl.ds(i * 16, 16)]
                ci = cidx_v[pl.ds(i * 16, 16)]
                bi = jnp.minimum(bi, jnp.where(w == msv, ci, bigv))
            miv = splat_min_i32(bi)
            wonv = msv > 0.0
            for i in range(NCV):
                w = cval_v[pl.ds(i * 16, 16)]
                ci = cidx_v[pl.ds(i * 16, 16)]
                cval_v[pl.ds(i * 16, 16)] = jnp.where(ci == miv, ninf, w)
            for g in range(NG):
                gx = chidx_v[pl.ds(g * 16, 16)]
                hit = jnp.logical_and(gx == miv, wonv)
                masks[g] = jnp.where(hit, jnp.float32(1.0), masks[g])

        for g in range(NG):
            mg = masks[g] > 0.0
            for t in range(T):
                x = pot_v[t, pl.ds(g * 16, 16)]
                out_v[t, pl.ds(g * 16, 16)] = jnp.where(
                    jnp.logical_and(x > 0.0, mg), 1.0, 0.0)
        pltpu.sync_copy(out_v, out_hbm.at[:, pl.ds(base, CH_PER)])


_sc_epilogue = functools.partial(
    pl.kernel,
    out_type=jax.ShapeDtypeStruct((T, K), jnp.float32),
    mesh=plsc.VectorSubcoreMesh(core_axis_name="c", subcore_axis_name="s"),
    compiler_params=pltpu.CompilerParams(needs_layout_passes=False),
    scratch_types=[
        pltpu.VMEM((T, CH_PER), jnp.float32),
        pltpu.VMEM((CH_PER,), jnp.int32),
        pltpu.VMEM((CH_PER,), jnp.float32),
        pltpu.VMEM((16,), jnp.float32),
        pltpu.VMEM((16,), jnp.float32),
        pltpu.VMEM((16,), jnp.int32),
        pltpu.VMEM((T, CH_PER), jnp.float32),
        pltpu.VMEM((NCAND,), jnp.float32),
        pltpu.VMEM((NCAND,), jnp.int32),
        pltpu.VMEM((16,), jnp.float32),
        pltpu.VMEM((16,), jnp.int32),
        pltpu.VMEM_SHARED((SC_SUB * 16,), jnp.float32),
        pltpu.VMEM_SHARED((NCAND,), jnp.float32),
        pltpu.VMEM_SHARED((NCAND,), jnp.int32),
    ],
)(_sc_epilogue_body)


@jax.jit
def kernel(rec_field, W):
    A = jnp.squeeze(rec_field, 1)
    Wm = jnp.squeeze(W, 1)

    pot = pl.pallas_call(
        _matmul_kernel,
        grid=(KC,),
        in_specs=[
            pl.BlockSpec((T, HBLK, L), lambda c: (0, c, 0)),
            pl.BlockSpec((K, HBLK, L), lambda c: (0, c, 0)),
        ],
        out_specs=pl.BlockSpec((T, K), lambda c: (0, 0)),
        out_shape=jax.ShapeDtypeStruct((T, K), jnp.float32),
        scratch_shapes=[pltpu.VMEM((T, K), jnp.float32)],
    )(A, Wm)

    chidx = jnp.arange(K, dtype=jnp.int32)
    spikes_masked = _sc_epilogue(pot, chidx)

    return spikes_masked.reshape(T, K, 1, 1)
